# initial kernel scaffold (unmeasured)
import jax
import jax.numpy as jnp
from jax import lax
from jax.experimental import pallas as pl
from jax.experimental.pallas import tpu as pltpu

B, S, H, Dh, Dr = 4, 256, 32, 128, 64
D = 4096
DC_HALF = 128
BS = B * S
SCALE = (Dh + Dr) ** -0.5

F32 = jnp.float32


def _c_and_exchange(x2d, Wdkv, Wuk, Wuv):

    def body(x_ref, wdkv_ref, wuk_ref, wuv_ref,
             c_ref, co_ref, wuko_ref, wuvo_ref,
             send_sems, recv_sems):
        my_x = lax.axis_index("x")
        my_y = lax.axis_index("y")
        my_z = lax.axis_index("z")
        peer = (my_x, 1 - my_y, my_z)

        barrier = pltpu.get_barrier_semaphore()
        pl.semaphore_signal(barrier, inc=1, device_id=peer,
                            device_id_type=pl.DeviceIdType.MESH)
        pl.semaphore_wait(barrier, 1)

        c_ref[...] = jnp.dot(x_ref[...], wdkv_ref[...],
                             preferred_element_type=F32)

        rdmas = []
        pairs = [(c_ref, co_ref), (wuk_ref, wuko_ref), (wuv_ref, wuvo_ref)]
        for i, (src, dst) in enumerate(pairs):
            rdma = pltpu.make_async_remote_copy(
                src_ref=src, dst_ref=dst,
                send_sem=send_sems.at[i], recv_sem=recv_sems.at[i],
                device_id=peer, device_id_type=pl.DeviceIdType.MESH,
            )
            rdma.start()
            rdmas.append(rdma)
        for rdma in rdmas:
            rdma.wait()

    return pl.pallas_call(
        body,
        out_shape=[
            jax.ShapeDtypeStruct((BS, DC_HALF), F32),
            jax.ShapeDtypeStruct((BS, DC_HALF), F32),
            jax.ShapeDtypeStruct((DC_HALF, D), F32),
            jax.ShapeDtypeStruct((DC_HALF, D), F32),
        ],
        in_specs=[pl.BlockSpec(memory_space=pltpu.VMEM)] * 4,
        out_specs=[pl.BlockSpec(memory_space=pltpu.VMEM)] * 4,
        scratch_shapes=[
            pltpu.SemaphoreType.DMA((3,)),
            pltpu.SemaphoreType.DMA((3,)),
        ],
        compiler_params=pltpu.CompilerParams(collective_id=0),
    )(x2d, Wdkv, Wuk, Wuv)


def _kv(c, co, Wuk, Wuko, Wuv, Wuvo):
    nb = 8
    blk = D // nb

    def body(c_ref, co_ref, wuk_ref, wuko_ref, wuv_ref, wuvo_ref,
             k_ref, v_ref):
        c_ = c_ref[...]
        co_ = co_ref[...]
        k_ref[...] = (jnp.dot(c_, wuk_ref[...], preferred_element_type=F32)
                      + jnp.dot(co_, wuko_ref[...], preferred_element_type=F32))
        v_ref[...] = (jnp.dot(c_, wuv_ref[...], preferred_element_type=F32)
                      + jnp.dot(co_, wuvo_ref[...], preferred_element_type=F32))

    return pl.pallas_call(
        body,
        grid=(nb,),
        in_specs=[
            pl.BlockSpec((BS, DC_HALF), lambda j: (0, 0)),
            pl.BlockSpec((BS, DC_HALF), lambda j: (0, 0)),
            pl.BlockSpec((DC_HALF, blk), lambda j: (0, j)),
            pl.BlockSpec((DC_HALF, blk), lambda j: (0, j)),
            pl.BlockSpec((DC_HALF, blk), lambda j: (0, j)),
            pl.BlockSpec((DC_HALF, blk), lambda j: (0, j)),
        ],
        out_specs=[
            pl.BlockSpec((BS, blk), lambda j: (0, j)),
            pl.BlockSpec((BS, blk), lambda j: (0, j)),
        ],
        out_shape=[
            jax.ShapeDtypeStruct((BS, D), F32),
            jax.ShapeDtypeStruct((BS, D), F32),
        ],
    )(c, co, Wuk, Wuko, Wuv, Wuvo)


def _matmul(a, w, n_blk):
    m, k = a.shape
    _, n = w.shape
    nb = n // n_blk

    def body(a_ref, w_ref, o_ref):
        o_ref[...] = jnp.dot(a_ref[...], w_ref[...],
                             preferred_element_type=F32)

    return pl.pallas_call(
        body,
        grid=(nb,),
        in_specs=[
            pl.BlockSpec((m, k), lambda j: (0, 0)),
            pl.BlockSpec((k, n_blk), lambda j: (0, j)),
        ],
        out_specs=pl.BlockSpec((m, n_blk), lambda j: (0, j)),
        out_shape=jax.ShapeDtypeStruct((m, n), F32),
    )(a, w)


def _attention(Q4, K4, V4, Qr4, Kr3):

    def body(q_ref, k_ref, v_ref, qr_ref, kr_ref, o_ref):
        q = q_ref[0, :, 0, :]
        k = k_ref[0, :, 0, :]
        v = v_ref[0, :, 0, :]
        qr = qr_ref[0, :, 0, :]
        kr = kr_ref[0, :, :]
        dn = (((1,), (1,)), ((), ()))
        s = (lax.dot_general(q, k, dn, preferred_element_type=F32)
             + lax.dot_general(qr, kr, dn, preferred_element_type=F32))
        s = s * SCALE
        m = jnp.max(s, axis=-1, keepdims=True)
        p = jnp.exp(s - m)
        p = p / jnp.sum(p, axis=-1, keepdims=True)
        o_ref[0, :, 0, :] = jnp.dot(p, v, preferred_element_type=F32)

    return pl.pallas_call(
        body,
        grid=(B, H),
        in_specs=[
            pl.BlockSpec((1, S, 1, Dh), lambda b, h: (b, 0, h, 0)),
            pl.BlockSpec((1, S, 1, Dh), lambda b, h: (b, 0, h, 0)),
            pl.BlockSpec((1, S, 1, Dh), lambda b, h: (b, 0, h, 0)),
            pl.BlockSpec((1, S, 1, Dr), lambda b, h: (b, 0, h, 0)),
            pl.BlockSpec((1, S, Dr), lambda b, h: (b, 0, 0)),
        ],
        out_specs=pl.BlockSpec((1, S, 1, Dh), lambda b, h: (b, 0, h, 0)),
        out_shape=jax.ShapeDtypeStruct((B, S, H, Dh), F32),
    )(Q4, K4, V4, Qr4, Kr3)


def kernel(x, Wdkv, Wuk, Wuv, Wq, Wqr, Wkr, Wo):
    x2d = x.reshape(BS, D)

    c, co, Wuko, Wuvo = _c_and_exchange(x2d, Wdkv, Wuk, Wuv)
    K, V = _kv(c, co, Wuk, Wuko, Wuv, Wuvo)

    Q = _matmul(x2d, Wq, 256)
    Qr = _matmul(x2d, Wqr, 256)
    Kr = _matmul(x2d, Wkr, 64)

    O = _attention(
        Q.reshape(B, S, H, Dh),
        K.reshape(B, S, H, Dh),
        V.reshape(B, S, H, Dh),
        Qr.reshape(B, S, H, Dr),
        Kr.reshape(B, S, Dr),
    )

    out = _matmul(O.reshape(BS, H * Dh), Wo, 256)
    return out.reshape(B, S, D)


# baseline (device time: 710317 ns/iter reference)
import jax
import jax.numpy as jnp
from jax import lax
from jax.experimental import pallas as pl
from jax.experimental.pallas import tpu as pltpu

B, S, H, Dh, Dr = 4, 256, 32, 128, 64
D = 4096
DC_HALF = 128
BS = B * S
SCALE = (Dh + Dr) ** -0.5

F32 = jnp.float32


def _c_and_exchange(x2d, Wdkv, Wuk, Wuv):

    def body(x_ref, wdkv_ref, wuk_ref, wuv_ref,
             c_ref, co_ref, wuko_ref, wuvo_ref,
             send_sems, recv_sems):
        my_x = lax.axis_index("x")
        my_y = lax.axis_index("y")
        my_z = lax.axis_index("z")
        peer = (my_x, 1 - my_y, my_z)

        barrier = pltpu.get_barrier_semaphore()
        pl.semaphore_signal(barrier, inc=1, device_id=peer,
                            device_id_type=pl.DeviceIdType.MESH)
        pl.semaphore_wait(barrier, 1)

        c_ref[...] = jnp.dot(x_ref[...], wdkv_ref[...],
                             preferred_element_type=F32)

        rdmas = []
        pairs = [(c_ref, co_ref), (wuk_ref, wuko_ref), (wuv_ref, wuvo_ref)]
        for i, (src, dst) in enumerate(pairs):
            rdma = pltpu.make_async_remote_copy(
                src_ref=src, dst_ref=dst,
                send_sem=send_sems.at[i], recv_sem=recv_sems.at[i],
                device_id=peer, device_id_type=pl.DeviceIdType.MESH,
            )
            rdma.start()
            rdmas.append(rdma)
        for rdma in rdmas:
            rdma.wait()

    return pl.pallas_call(
        body,
        out_shape=[
            jax.ShapeDtypeStruct((BS, DC_HALF), F32),
            jax.ShapeDtypeStruct((BS, DC_HALF), F32),
            jax.ShapeDtypeStruct((DC_HALF, D), F32),
            jax.ShapeDtypeStruct((DC_HALF, D), F32),
        ],
        in_specs=[pl.BlockSpec(memory_space=pltpu.VMEM)] * 4,
        out_specs=[pl.BlockSpec(memory_space=pltpu.VMEM)] * 4,
        scratch_shapes=[
            pltpu.SemaphoreType.DMA((3,)),
            pltpu.SemaphoreType.DMA((3,)),
        ],
        compiler_params=pltpu.CompilerParams(collective_id=0),
    )(x2d, Wdkv, Wuk, Wuv)


def _kv(c, co, Wuk, Wuko, Wuv, Wuvo):

    def body(c_ref, co_ref, wuk_ref, wuko_ref, wuv_ref, wuvo_ref,
             k_ref, v_ref):
        c_ = c_ref[...]
        co_ = co_ref[...]
        k_ref[0] = (jnp.dot(c_, wuk_ref[...], preferred_element_type=F32)
                    + jnp.dot(co_, wuko_ref[...], preferred_element_type=F32))
        v_ref[0] = (jnp.dot(c_, wuv_ref[...], preferred_element_type=F32)
                    + jnp.dot(co_, wuvo_ref[...], preferred_element_type=F32))

    return pl.pallas_call(
        body,
        grid=(H,),
        in_specs=[
            pl.BlockSpec((BS, DC_HALF), lambda h: (0, 0)),
            pl.BlockSpec((BS, DC_HALF), lambda h: (0, 0)),
            pl.BlockSpec((DC_HALF, Dh), lambda h: (0, h)),
            pl.BlockSpec((DC_HALF, Dh), lambda h: (0, h)),
            pl.BlockSpec((DC_HALF, Dh), lambda h: (0, h)),
            pl.BlockSpec((DC_HALF, Dh), lambda h: (0, h)),
        ],
        out_specs=[
            pl.BlockSpec((1, BS, Dh), lambda h: (h, 0, 0)),
            pl.BlockSpec((1, BS, Dh), lambda h: (h, 0, 0)),
        ],
        out_shape=[
            jax.ShapeDtypeStruct((H, BS, Dh), F32),
            jax.ShapeDtypeStruct((H, BS, Dh), F32),
        ],
    )(c, co, Wuk, Wuko, Wuv, Wuvo)


def _q_proj(x2d, Wq):

    def body(x_ref, w_ref, o_ref):
        o_ref[0] = jnp.dot(x_ref[...], w_ref[...], preferred_element_type=F32)

    return pl.pallas_call(
        body,
        grid=(H,),
        in_specs=[
            pl.BlockSpec((BS, D), lambda h: (0, 0)),
            pl.BlockSpec((D, Dh), lambda h: (0, h)),
        ],
        out_specs=pl.BlockSpec((1, BS, Dh), lambda h: (h, 0, 0)),
        out_shape=jax.ShapeDtypeStruct((H, BS, Dh), F32),
    )(x2d, Wq)


def _qr_proj(x2d, Wqr):

    def body(x_ref, w_ref, o_ref):
        r = jnp.dot(x_ref[...], w_ref[...], preferred_element_type=F32)
        o_ref[0] = r[:, :Dr]
        o_ref[1] = r[:, Dr:]

    return pl.pallas_call(
        body,
        grid=(H // 2,),
        in_specs=[
            pl.BlockSpec((BS, D), lambda j: (0, 0)),
            pl.BlockSpec((D, 2 * Dr), lambda j: (0, j)),
        ],
        out_specs=pl.BlockSpec((2, BS, Dr), lambda j: (j, 0, 0)),
        out_shape=jax.ShapeDtypeStruct((H, BS, Dr), F32),
    )(x2d, Wqr)


def _kr_proj(x2d, Wkr):

    def body(x_ref, w_ref, o_ref):
        o_ref[...] = jnp.dot(x_ref[...], w_ref[...], preferred_element_type=F32)

    return pl.pallas_call(
        body,
        in_specs=[pl.BlockSpec(memory_space=pltpu.VMEM)] * 2,
        out_specs=pl.BlockSpec(memory_space=pltpu.VMEM),
        out_shape=jax.ShapeDtypeStruct((BS, Dr), F32),
    )(x2d, Wkr)


def _attention(Q, Kt, V, Qr, Kr):

    def body(q_ref, k_ref, v_ref, qr_ref, kr_ref, o_ref):
        q = q_ref[0, 0]
        k = k_ref[0, 0]
        v = v_ref[0, 0]
        qr = qr_ref[0, 0]
        kr = kr_ref[0]
        dn = (((1,), (1,)), ((), ()))
        s = (lax.dot_general(q, k, dn, preferred_element_type=F32)
             + lax.dot_general(qr, kr, dn, preferred_element_type=F32))
        s = s * SCALE
        m = jnp.max(s, axis=-1, keepdims=True)
        p = jnp.exp(s - m)
        p = p / jnp.sum(p, axis=-1, keepdims=True)
        o_ref[0, 0] = jnp.dot(p, v, preferred_element_type=F32)

    return pl.pallas_call(
        body,
        grid=(H, B),
        in_specs=[
            pl.BlockSpec((1, 1, S, Dh), lambda h, b: (h, b, 0, 0)),
            pl.BlockSpec((1, 1, S, Dh), lambda h, b: (h, b, 0, 0)),
            pl.BlockSpec((1, 1, S, Dh), lambda h, b: (h, b, 0, 0)),
            pl.BlockSpec((1, 1, S, Dr), lambda h, b: (h, b, 0, 0)),
            pl.BlockSpec((1, S, Dr), lambda h, b: (b, 0, 0)),
        ],
        out_specs=pl.BlockSpec((1, 1, S, Dh), lambda h, b: (h, b, 0, 0)),
        out_shape=jax.ShapeDtypeStruct((H, B, S, Dh), F32),
    )(Q, Kt, V, Qr, Kr)


def _out_proj(O, Wo):
    nb = 16
    blk = D // nb

    def body(o_ref, w_ref, out_ref):
        h = pl.program_id(1)
        part = jnp.dot(o_ref[0], w_ref[...], preferred_element_type=F32)

        @pl.when(h == 0)
        def _():
            out_ref[...] = part

        @pl.when(h > 0)
        def _():
            out_ref[...] += part

    return pl.pallas_call(
        body,
        grid=(nb, H),
        in_specs=[
            pl.BlockSpec((1, BS, Dh), lambda j, h: (h, 0, 0)),
            pl.BlockSpec((Dh, blk), lambda j, h: (h, j)),
        ],
        out_specs=pl.BlockSpec((BS, blk), lambda j, h: (0, j)),
        out_shape=jax.ShapeDtypeStruct((BS, D), F32),
    )(O, Wo)


def kernel(x, Wdkv, Wuk, Wuv, Wq, Wqr, Wkr, Wo):
    x2d = x.reshape(BS, D)

    c, co, Wuko, Wuvo = _c_and_exchange(x2d, Wdkv, Wuk, Wuv)
    K, V = _kv(c, co, Wuk, Wuko, Wuv, Wuvo)

    Q = _q_proj(x2d, Wq)
    Qr = _qr_proj(x2d, Wqr)
    Kr = _kr_proj(x2d, Wkr)

    O = _attention(
        Q.reshape(H, B, S, Dh),
        K.reshape(H, B, S, Dh),
        V.reshape(H, B, S, Dh),
        Qr.reshape(H, B, S, Dr),
        Kr.reshape(B, S, Dr),
    )

    out = _out_proj(O.reshape(H, BS, Dh), Wo)
    return out.reshape(B, S, D)


# device time: 666782 ns/iter; 1.0653x vs baseline; 1.0653x over previous
import jax
import jax.numpy as jnp
from jax import lax
from jax.experimental import pallas as pl
from jax.experimental.pallas import tpu as pltpu

B, S, H, Dh, Dr = 4, 256, 32, 128, 64
D = 4096
DC_HALF = 128
BS = B * S
SCALE = (Dh + Dr) ** -0.5

F32 = jnp.float32
BF16 = jnp.bfloat16


def _c_and_exchange(x2d, Wdkv, Wuk, Wuv):

    def body(x_ref, wdkv_ref, wuk_ref, wuv_ref,
             xbf_ref, c_ref, co_ref, wuk_ref_o, wuko_ref, wuv_ref_o,
             wuvo_ref, send_sems, recv_sems):
        my_x = lax.axis_index("x")
        my_y = lax.axis_index("y")
        my_z = lax.axis_index("z")
        peer = (my_x, 1 - my_y, my_z)

        barrier = pltpu.get_barrier_semaphore()
        pl.semaphore_signal(barrier, inc=1, device_id=peer,
                            device_id_type=pl.DeviceIdType.MESH)
        pl.semaphore_wait(barrier, 1)

        xbf_ref[...] = x_ref[...].astype(BF16)
        wuk_ref_o[...] = wuk_ref[...].astype(BF16)
        wuv_ref_o[...] = wuv_ref[...].astype(BF16)
        c_ref[...] = jnp.dot(
            xbf_ref[...], wdkv_ref[...].astype(BF16),
            preferred_element_type=F32,
        ).astype(BF16)

        rdmas = []
        pairs = [(c_ref, co_ref), (wuk_ref_o, wuko_ref), (wuv_ref_o, wuvo_ref)]
        for i, (src, dst) in enumerate(pairs):
            rdma = pltpu.make_async_remote_copy(
                src_ref=src, dst_ref=dst,
                send_sem=send_sems.at[i], recv_sem=recv_sems.at[i],
                device_id=peer, device_id_type=pl.DeviceIdType.MESH,
            )
            rdma.start()
            rdmas.append(rdma)
        for rdma in rdmas:
            rdma.wait()

    return pl.pallas_call(
        body,
        out_shape=[
            jax.ShapeDtypeStruct((BS, D), BF16),
            jax.ShapeDtypeStruct((BS, DC_HALF), BF16),
            jax.ShapeDtypeStruct((BS, DC_HALF), BF16),
            jax.ShapeDtypeStruct((DC_HALF, D), BF16),
            jax.ShapeDtypeStruct((DC_HALF, D), BF16),
            jax.ShapeDtypeStruct((DC_HALF, D), BF16),
            jax.ShapeDtypeStruct((DC_HALF, D), BF16),
        ],
        in_specs=[pl.BlockSpec(memory_space=pltpu.VMEM)] * 4,
        out_specs=[pl.BlockSpec(memory_space=pltpu.VMEM)] * 7,
        scratch_shapes=[
            pltpu.SemaphoreType.DMA((3,)),
            pltpu.SemaphoreType.DMA((3,)),
        ],
        compiler_params=pltpu.CompilerParams(collective_id=0),
    )(x2d, Wdkv, Wuk, Wuv)


def _kv(c, co, Wuk, Wuko, Wuv, Wuvo):

    def body(c_ref, co_ref, wuk_ref, wuko_ref, wuv_ref, wuvo_ref,
             k_ref, v_ref):
        c_ = c_ref[...]
        co_ = co_ref[...]
        k_ref[0] = (jnp.dot(c_, wuk_ref[...], preferred_element_type=F32)
                    + jnp.dot(co_, wuko_ref[...], preferred_element_type=F32)
                    ).astype(BF16)
        v_ref[0] = (jnp.dot(c_, wuv_ref[...], preferred_element_type=F32)
                    + jnp.dot(co_, wuvo_ref[...], preferred_element_type=F32)
                    ).astype(BF16)

    return pl.pallas_call(
        body,
        grid=(H,),
        in_specs=[
            pl.BlockSpec((BS, DC_HALF), lambda h: (0, 0)),
            pl.BlockSpec((BS, DC_HALF), lambda h: (0, 0)),
            pl.BlockSpec((DC_HALF, Dh), lambda h: (0, h)),
            pl.BlockSpec((DC_HALF, Dh), lambda h: (0, h)),
            pl.BlockSpec((DC_HALF, Dh), lambda h: (0, h)),
            pl.BlockSpec((DC_HALF, Dh), lambda h: (0, h)),
        ],
        out_specs=[
            pl.BlockSpec((1, BS, Dh), lambda h: (h, 0, 0)),
            pl.BlockSpec((1, BS, Dh), lambda h: (h, 0, 0)),
        ],
        out_shape=[
            jax.ShapeDtypeStruct((H, BS, Dh), BF16),
            jax.ShapeDtypeStruct((H, BS, Dh), BF16),
        ],
    )(c, co, Wuk, Wuko, Wuv, Wuvo)


def _q_proj(xbf, Wq):

    def body(x_ref, w_ref, o_ref):
        o_ref[0] = jnp.dot(x_ref[...], w_ref[...].astype(BF16),
                           preferred_element_type=F32).astype(BF16)

    return pl.pallas_call(
        body,
        grid=(H,),
        in_specs=[
            pl.BlockSpec((BS, D), lambda h: (0, 0)),
            pl.BlockSpec((D, Dh), lambda h: (0, h)),
        ],
        out_specs=pl.BlockSpec((1, BS, Dh), lambda h: (h, 0, 0)),
        out_shape=jax.ShapeDtypeStruct((H, BS, Dh), BF16),
    )(xbf, Wq)


def _qr_proj(xbf, Wqr):

    def body(x_ref, w_ref, o_ref):
        r = jnp.dot(x_ref[...], w_ref[...].astype(BF16),
                    preferred_element_type=F32).astype(BF16)
        o_ref[0] = r[:, :Dr]
        o_ref[1] = r[:, Dr:]

    return pl.pallas_call(
        body,
        grid=(H // 2,),
        in_specs=[
            pl.BlockSpec((BS, D), lambda j: (0, 0)),
            pl.BlockSpec((D, 2 * Dr), lambda j: (0, j)),
        ],
        out_specs=pl.BlockSpec((2, BS, Dr), lambda j: (j, 0, 0)),
        out_shape=jax.ShapeDtypeStruct((H, BS, Dr), BF16),
    )(xbf, Wqr)


def _kr_proj(xbf, Wkr):

    def body(x_ref, w_ref, o_ref):
        o_ref[...] = jnp.dot(x_ref[...], w_ref[...].astype(BF16),
                             preferred_element_type=F32).astype(BF16)

    return pl.pallas_call(
        body,
        in_specs=[pl.BlockSpec(memory_space=pltpu.VMEM)] * 2,
        out_specs=pl.BlockSpec(memory_space=pltpu.VMEM),
        out_shape=jax.ShapeDtypeStruct((BS, Dr), BF16),
    )(xbf, Wkr)


def _attention(Q, Kt, V, Qr, Kr):

    def body(q_ref, k_ref, v_ref, qr_ref, kr_ref, o_ref):
        q = q_ref[0, 0]
        k = k_ref[0, 0]
        v = v_ref[0, 0]
        qr = qr_ref[0, 0]
        kr = kr_ref[0]
        dn = (((1,), (1,)), ((), ()))
        s = (lax.dot_general(q, k, dn, preferred_element_type=F32)
             + lax.dot_general(qr, kr, dn, preferred_element_type=F32))
        s = s * SCALE
        m = jnp.max(s, axis=-1, keepdims=True)
        p = jnp.exp(s - m)
        p = p / jnp.sum(p, axis=-1, keepdims=True)
        o_ref[0, 0] = jnp.dot(p.astype(BF16), v,
                              preferred_element_type=F32).astype(BF16)

    return pl.pallas_call(
        body,
        grid=(H, B),
        in_specs=[
            pl.BlockSpec((1, 1, S, Dh), lambda h, b: (h, b, 0, 0)),
            pl.BlockSpec((1, 1, S, Dh), lambda h, b: (h, b, 0, 0)),
            pl.BlockSpec((1, 1, S, Dh), lambda h, b: (h, b, 0, 0)),
            pl.BlockSpec((1, 1, S, Dr), lambda h, b: (h, b, 0, 0)),
            pl.BlockSpec((1, S, Dr), lambda h, b: (b, 0, 0)),
        ],
        out_specs=pl.BlockSpec((1, 1, S, Dh), lambda h, b: (h, b, 0, 0)),
        out_shape=jax.ShapeDtypeStruct((H, B, S, Dh), BF16),
    )(Q, Kt, V, Qr, Kr)


def _out_proj(O, Wo):
    nb = 16
    blk = D // nb

    def body(o_ref, w_ref, out_ref):
        h = pl.program_id(1)
        part = jnp.dot(o_ref[0], w_ref[...].astype(BF16),
                       preferred_element_type=F32)

        @pl.when(h == 0)
        def _():
            out_ref[...] = part

        @pl.when(h > 0)
        def _():
            out_ref[...] += part

    return pl.pallas_call(
        body,
        grid=(nb, H),
        in_specs=[
            pl.BlockSpec((1, BS, Dh), lambda j, h: (h, 0, 0)),
            pl.BlockSpec((Dh, blk), lambda j, h: (h, j)),
        ],
        out_specs=pl.BlockSpec((BS, blk), lambda j, h: (0, j)),
        out_shape=jax.ShapeDtypeStruct((BS, D), F32),
    )(O, Wo)


def kernel(x, Wdkv, Wuk, Wuv, Wq, Wqr, Wkr, Wo):
    x2d = x.reshape(BS, D)

    xbf, c, co, Wukb, Wuko, Wuvb, Wuvo = _c_and_exchange(x2d, Wdkv, Wuk, Wuv)
    K, V = _kv(c, co, Wukb, Wuko, Wuvb, Wuvo)

    Q = _q_proj(xbf, Wq)
    Qr = _qr_proj(xbf, Wqr)
    Kr = _kr_proj(xbf, Wkr)

    O = _attention(
        Q.reshape(H, B, S, Dh),
        K.reshape(H, B, S, Dh),
        V.reshape(H, B, S, Dh),
        Qr.reshape(H, B, S, Dr),
        Kr.reshape(B, S, Dr),
    )

    out = _out_proj(O.reshape(H, BS, Dh), Wo)
    return out.reshape(B, S, D)


# device time: 264800 ns/iter; 2.6825x vs baseline; 2.5181x over previous
import jax
import jax.numpy as jnp
from jax import lax
from jax.experimental import pallas as pl
from jax.experimental.pallas import tpu as pltpu

B, S, H, Dh, Dr = 4, 256, 32, 128, 64
D = 4096
DC_HALF = 128
BS = B * S
SCALE = (Dh + Dr) ** -0.5

F32 = jnp.float32
BF16 = jnp.bfloat16


def _c_and_exchange(x2d, Wdkv, Wuk, Wuv):

    def body(x_ref, wdkv_ref, wuk_ref, wuv_ref,
             xbf_ref, c_ref, co_ref, wuk_ref_o, wuko_ref, wuv_ref_o,
             wuvo_ref, send_sems, recv_sems):
        my_x = lax.axis_index("x")
        my_y = lax.axis_index("y")
        my_z = lax.axis_index("z")
        peer = (my_x, 1 - my_y, my_z)

        barrier = pltpu.get_barrier_semaphore()
        pl.semaphore_signal(barrier, inc=1, device_id=peer,
                            device_id_type=pl.DeviceIdType.MESH)
        pl.semaphore_wait(barrier, 1)

        xbf_ref[...] = x_ref[...].astype(BF16)
        wuk_ref_o[...] = wuk_ref[...].astype(BF16)
        wuv_ref_o[...] = wuv_ref[...].astype(BF16)
        c_ref[...] = jnp.dot(
            xbf_ref[...], wdkv_ref[...].astype(BF16),
            preferred_element_type=F32,
        ).astype(BF16)

        rdmas = []
        pairs = [(c_ref, co_ref), (wuk_ref_o, wuko_ref), (wuv_ref_o, wuvo_ref)]
        for i, (src, dst) in enumerate(pairs):
            rdma = pltpu.make_async_remote_copy(
                src_ref=src, dst_ref=dst,
                send_sem=send_sems.at[i], recv_sem=recv_sems.at[i],
                device_id=peer, device_id_type=pl.DeviceIdType.MESH,
            )
            rdma.start()
            rdmas.append(rdma)
        for rdma in rdmas:
            rdma.wait()

    return pl.pallas_call(
        body,
        out_shape=[
            jax.ShapeDtypeStruct((BS, D), BF16),
            jax.ShapeDtypeStruct((BS, DC_HALF), BF16),
            jax.ShapeDtypeStruct((BS, DC_HALF), BF16),
            jax.ShapeDtypeStruct((DC_HALF, D), BF16),
            jax.ShapeDtypeStruct((DC_HALF, D), BF16),
            jax.ShapeDtypeStruct((DC_HALF, D), BF16),
            jax.ShapeDtypeStruct((DC_HALF, D), BF16),
        ],
        in_specs=[pl.BlockSpec(memory_space=pltpu.VMEM)] * 4,
        out_specs=[pl.BlockSpec(memory_space=pltpu.VMEM)] * 7,
        scratch_shapes=[
            pltpu.SemaphoreType.DMA((3,)),
            pltpu.SemaphoreType.DMA((3,)),
        ],
        compiler_params=pltpu.CompilerParams(collective_id=0),
    )(x2d, Wdkv, Wuk, Wuv)


def _kv(c, co, Wuk, Wuko, Wuv, Wuvo):
    nb = 16
    blk = D // nb

    def body(c_ref, co_ref, wuk_ref, wuko_ref, wuv_ref, wuvo_ref,
             k_ref, v_ref):
        c_ = c_ref[...]
        co_ = co_ref[...]
        k_ref[...] = (jnp.dot(c_, wuk_ref[...], preferred_element_type=F32)
                      + jnp.dot(co_, wuko_ref[...], preferred_element_type=F32)
                      ).astype(BF16)
        v_ref[...] = (jnp.dot(c_, wuv_ref[...], preferred_element_type=F32)
                      + jnp.dot(co_, wuvo_ref[...], preferred_element_type=F32)
                      ).astype(BF16)

    return pl.pallas_call(
        body,
        grid=(nb,),
        in_specs=[
            pl.BlockSpec((BS, DC_HALF), lambda j: (0, 0)),
            pl.BlockSpec((BS, DC_HALF), lambda j: (0, 0)),
            pl.BlockSpec((DC_HALF, blk), lambda j: (0, j)),
            pl.BlockSpec((DC_HALF, blk), lambda j: (0, j)),
            pl.BlockSpec((DC_HALF, blk), lambda j: (0, j)),
            pl.BlockSpec((DC_HALF, blk), lambda j: (0, j)),
        ],
        out_specs=[
            pl.BlockSpec((BS, blk), lambda j: (0, j)),
            pl.BlockSpec((BS, blk), lambda j: (0, j)),
        ],
        out_shape=[
            jax.ShapeDtypeStruct((BS, D), BF16),
            jax.ShapeDtypeStruct((BS, D), BF16),
        ],
    )(c, co, Wuk, Wuko, Wuv, Wuvo)


def _matmul(a_bf, w, n_blk, out_dtype):
    m, k = a_bf.shape
    _, n = w.shape
    nb = n // n_blk

    def body(a_ref, w_ref, o_ref):
        o_ref[...] = jnp.dot(
            a_ref[...], w_ref[...].astype(BF16),
            preferred_element_type=F32,
        ).astype(out_dtype)

    return pl.pallas_call(
        body,
        grid=(nb,),
        in_specs=[
            pl.BlockSpec((m, k), lambda j: (0, 0)),
            pl.BlockSpec((k, n_blk), lambda j: (0, j)),
        ],
        out_specs=pl.BlockSpec((m, n_blk), lambda j: (0, j)),
        out_shape=jax.ShapeDtypeStruct((m, n), out_dtype),
    )(a_bf, w)


def _qr_proj(xbf, Wqr):

    def body(x_ref, w_ref, o_ref):
        r = jnp.dot(x_ref[...], w_ref[...].astype(BF16),
                    preferred_element_type=F32).astype(BF16)
        o_ref[0] = r[:, :Dr]
        o_ref[1] = r[:, Dr:]

    return pl.pallas_call(
        body,
        grid=(H // 2,),
        in_specs=[
            pl.BlockSpec((BS, D), lambda j: (0, 0)),
            pl.BlockSpec((D, 2 * Dr), lambda j: (0, j)),
        ],
        out_specs=pl.BlockSpec((2, BS, Dr), lambda j: (j, 0, 0)),
        out_shape=jax.ShapeDtypeStruct((H, BS, Dr), BF16),
    )(xbf, Wqr)


def _kr_proj(xbf, Wkr):

    def body(x_ref, w_ref, o_ref):
        o_ref[...] = jnp.dot(x_ref[...], w_ref[...].astype(BF16),
                             preferred_element_type=F32).astype(BF16)

    return pl.pallas_call(
        body,
        in_specs=[pl.BlockSpec(memory_space=pltpu.VMEM)] * 2,
        out_specs=pl.BlockSpec(memory_space=pltpu.VMEM),
        out_shape=jax.ShapeDtypeStruct((BS, Dr), BF16),
    )(xbf, Wkr)


def _attention(Q2, K2, V2, Qr, Kr):

    def body(q_ref, k_ref, v_ref, qr_ref, kr_ref, o_ref):
        dn = (((1,), (1,)), ((), ()))
        for b in range(B):
            r = slice(b * S, (b + 1) * S)
            q = q_ref[r, :]
            k = k_ref[r, :]
            v = v_ref[r, :]
            qr = qr_ref[0, r, :]
            kr = kr_ref[r, :]
            s = (lax.dot_general(q, k, dn, preferred_element_type=F32)
                 + lax.dot_general(qr, kr, dn, preferred_element_type=F32))
            s = s * SCALE
            m = jnp.max(s, axis=-1, keepdims=True)
            p = jnp.exp(s - m)
            p = p / jnp.sum(p, axis=-1, keepdims=True)
            o_ref[r, :] = jnp.dot(p.astype(BF16), v,
                                  preferred_element_type=F32).astype(BF16)

    return pl.pallas_call(
        body,
        grid=(H,),
        in_specs=[
            pl.BlockSpec((BS, Dh), lambda h: (0, h)),
            pl.BlockSpec((BS, Dh), lambda h: (0, h)),
            pl.BlockSpec((BS, Dh), lambda h: (0, h)),
            pl.BlockSpec((1, BS, Dr), lambda h: (h, 0, 0)),
            pl.BlockSpec((BS, Dr), lambda h: (0, 0)),
        ],
        out_specs=pl.BlockSpec((BS, Dh), lambda h: (0, h)),
        out_shape=jax.ShapeDtypeStruct((BS, D), BF16),
    )(Q2, K2, V2, Qr, Kr)


def kernel(x, Wdkv, Wuk, Wuv, Wq, Wqr, Wkr, Wo):
    x2d = x.reshape(BS, D)

    xbf, c, co, Wukb, Wuko, Wuvb, Wuvo = _c_and_exchange(x2d, Wdkv, Wuk, Wuv)
    K, V = _kv(c, co, Wukb, Wuko, Wuvb, Wuvo)

    Q = _matmul(xbf, Wq, 512, BF16)
    Qr = _qr_proj(xbf, Wqr)
    Kr = _kr_proj(xbf, Wkr)

    O = _attention(Q, K, V, Qr, Kr)

    out = _matmul(O, Wo, 512, F32)
    return out.reshape(B, S, D)


# device time: 257809 ns/iter; 2.7552x vs baseline; 1.0271x over previous
import jax
import jax.numpy as jnp
from jax import lax
from jax.experimental import pallas as pl
from jax.experimental.pallas import tpu as pltpu

B, S, H, Dh, Dr = 4, 256, 32, 128, 64
D = 4096
DC_HALF = 128
BS = B * S
SCALE = (Dh + Dr) ** -0.5

F32 = jnp.float32
BF16 = jnp.bfloat16


def _c_and_exchange(x2d, Wdkv, Wuk, Wuv):

    def body(x_ref, wdkv_ref, wuk_ref, wuv_ref,
             xbf_ref, cc_ref, wukf_ref, wuvf_ref,
             send_sems, recv_sems):
        my_x = lax.axis_index("x")
        my_y = lax.axis_index("y")
        my_z = lax.axis_index("z")
        peer = (my_x, 1 - my_y, my_z)

        barrier = pltpu.get_barrier_semaphore()
        pl.semaphore_signal(barrier, inc=1, device_id=peer,
                            device_id_type=pl.DeviceIdType.MESH)
        pl.semaphore_wait(barrier, 1)

        xbf_ref[...] = x_ref[...].astype(BF16)
        my_c = jnp.dot(
            xbf_ref[...], wdkv_ref[...].astype(BF16),
            preferred_element_type=F32,
        ).astype(BF16)

        def exchange(lo, hi):
            cc_ref[:, lo:hi] = my_c
            wukf_ref[lo:hi, :] = wuk_ref[...].astype(BF16)
            wuvf_ref[lo:hi, :] = wuv_ref[...].astype(BF16)
            rdmas = []
            srcs = [cc_ref.at[:, lo:hi], wukf_ref.at[lo:hi, :],
                    wuvf_ref.at[lo:hi, :]]
            for i, src in enumerate(srcs):
                rdma = pltpu.make_async_remote_copy(
                    src_ref=src, dst_ref=src,
                    send_sem=send_sems.at[i], recv_sem=recv_sems.at[i],
                    device_id=peer, device_id_type=pl.DeviceIdType.MESH,
                )
                rdma.start()
                rdmas.append(rdma)
            for rdma in rdmas:
                rdma.wait()

        @pl.when(my_y == 0)
        def _():
            exchange(0, DC_HALF)

        @pl.when(my_y == 1)
        def _():
            exchange(DC_HALF, 2 * DC_HALF)

    return pl.pallas_call(
        body,
        out_shape=[
            jax.ShapeDtypeStruct((BS, D), BF16),
            jax.ShapeDtypeStruct((BS, 2 * DC_HALF), BF16),
            jax.ShapeDtypeStruct((2 * DC_HALF, D), BF16),
            jax.ShapeDtypeStruct((2 * DC_HALF, D), BF16),
        ],
        in_specs=[pl.BlockSpec(memory_space=pltpu.VMEM)] * 4,
        out_specs=[pl.BlockSpec(memory_space=pltpu.VMEM)] * 4,
        scratch_shapes=[
            pltpu.SemaphoreType.DMA((3,)),
            pltpu.SemaphoreType.DMA((3,)),
        ],
        compiler_params=pltpu.CompilerParams(collective_id=0),
    )(x2d, Wdkv, Wuk, Wuv)


def _kv(cc, Wukf, Wuvf):
    nb = 8
    blk = D // nb
    dc = 2 * DC_HALF

    def body(cc_ref, wuk_ref, wuv_ref, k_ref, v_ref):
        c_ = cc_ref[...]
        k_ref[...] = jnp.dot(c_, wuk_ref[...],
                             preferred_element_type=F32).astype(BF16)
        v_ref[...] = jnp.dot(c_, wuv_ref[...],
                             preferred_element_type=F32).astype(BF16)

    return pl.pallas_call(
        body,
        grid=(nb,),
        in_specs=[
            pl.BlockSpec((BS, dc), lambda j: (0, 0)),
            pl.BlockSpec((dc, blk), lambda j: (0, j)),
            pl.BlockSpec((dc, blk), lambda j: (0, j)),
        ],
        out_specs=[
            pl.BlockSpec((BS, blk), lambda j: (0, j)),
            pl.BlockSpec((BS, blk), lambda j: (0, j)),
        ],
        out_shape=[
            jax.ShapeDtypeStruct((BS, D), BF16),
            jax.ShapeDtypeStruct((BS, D), BF16),
        ],
    )(cc, Wukf, Wuvf)


def _matmul(a_bf, w, n_blk, out_dtype):
    m, k = a_bf.shape
    _, n = w.shape
    nb = n // n_blk

    def body(a_ref, w_ref, o_ref):
        o_ref[...] = jnp.dot(
            a_ref[...], w_ref[...].astype(BF16),
            preferred_element_type=F32,
        ).astype(out_dtype)

    return pl.pallas_call(
        body,
        grid=(nb,),
        in_specs=[
            pl.BlockSpec((m, k), lambda j: (0, 0)),
            pl.BlockSpec((k, n_blk), lambda j: (0, j)),
        ],
        out_specs=pl.BlockSpec((m, n_blk), lambda j: (0, j)),
        out_shape=jax.ShapeDtypeStruct((m, n), out_dtype),
    )(a_bf, w)


def _qr_proj(xbf, Wqr):

    def body(x_ref, w_ref, o_ref):
        r = jnp.dot(x_ref[...], w_ref[...].astype(BF16),
                    preferred_element_type=F32).astype(BF16)
        o_ref[0] = r[:, :Dr]
        o_ref[1] = r[:, Dr:]

    return pl.pallas_call(
        body,
        grid=(H // 2,),
        in_specs=[
            pl.BlockSpec((BS, D), lambda j: (0, 0)),
            pl.BlockSpec((D, 2 * Dr), lambda j: (0, j)),
        ],
        out_specs=pl.BlockSpec((2, BS, Dr), lambda j: (j, 0, 0)),
        out_shape=jax.ShapeDtypeStruct((H, BS, Dr), BF16),
    )(xbf, Wqr)


def _kr_proj(xbf, Wkr):

    def body(x_ref, w_ref, o_ref):
        o_ref[...] = jnp.dot(x_ref[...], w_ref[...].astype(BF16),
                             preferred_element_type=F32).astype(BF16)

    return pl.pallas_call(
        body,
        in_specs=[pl.BlockSpec(memory_space=pltpu.VMEM)] * 2,
        out_specs=pl.BlockSpec(memory_space=pltpu.VMEM),
        out_shape=jax.ShapeDtypeStruct((BS, Dr), BF16),
    )(xbf, Wkr)


def _attention(Q2, K2, V2, Qr, Kr):

    def body(q_ref, k_ref, v_ref, qr_ref, kr_ref, o_ref):
        dn = (((1,), (1,)), ((), ()))
        for b in range(B):
            r = slice(b * S, (b + 1) * S)
            q = q_ref[r, :]
            k = k_ref[r, :]
            v = v_ref[r, :]
            qr = qr_ref[0, r, :]
            kr = kr_ref[r, :]
            s = (lax.dot_general(q, k, dn, preferred_element_type=F32)
                 + lax.dot_general(qr, kr, dn, preferred_element_type=F32))
            s = s * SCALE
            m = jnp.max(s, axis=-1, keepdims=True)
            p = jnp.exp(s - m)
            p = p / jnp.sum(p, axis=-1, keepdims=True)
            o_ref[r, :] = jnp.dot(p.astype(BF16), v,
                                  preferred_element_type=F32).astype(BF16)

    return pl.pallas_call(
        body,
        grid=(H,),
        in_specs=[
            pl.BlockSpec((BS, Dh), lambda h: (0, h)),
            pl.BlockSpec((BS, Dh), lambda h: (0, h)),
            pl.BlockSpec((BS, Dh), lambda h: (0, h)),
            pl.BlockSpec((1, BS, Dr), lambda h: (h, 0, 0)),
            pl.BlockSpec((BS, Dr), lambda h: (0, 0)),
        ],
        out_specs=pl.BlockSpec((BS, Dh), lambda h: (0, h)),
        out_shape=jax.ShapeDtypeStruct((BS, D), BF16),
    )(Q2, K2, V2, Qr, Kr)


def kernel(x, Wdkv, Wuk, Wuv, Wq, Wqr, Wkr, Wo):
    x2d = x.reshape(BS, D)

    xbf, cc, Wukf, Wuvf = _c_and_exchange(x2d, Wdkv, Wuk, Wuv)
    K, V = _kv(cc, Wukf, Wuvf)

    Q = _matmul(xbf, Wq, 512, BF16)
    Qr = _qr_proj(xbf, Wqr)
    Kr = _kr_proj(xbf, Wkr)

    O = _attention(Q, K, V, Qr, Kr)

    out = _matmul(O, Wo, 512, F32)
    return out.reshape(B, S, D)


# device time: 239634 ns/iter; 2.9642x vs baseline; 1.0758x over previous
import jax
import jax.numpy as jnp
from jax import lax
from jax.experimental import pallas as pl
from jax.experimental.pallas import tpu as pltpu

B, S, H, Dh, Dr = 4, 256, 32, 128, 64
D = 4096
DC_HALF = 128
BS = B * S
SCALE = (Dh + Dr) ** -0.5

F32 = jnp.float32
BF16 = jnp.bfloat16


def _c_and_exchange(x2d, Wdkv, Wuk, Wuv):

    def body(x_ref, wdkv_ref, wuk_ref, wuv_ref,
             xbf_ref, cc_ref, wukf_ref, wuvf_ref,
             send_sems, recv_sems):
        my_x = lax.axis_index("x")
        my_y = lax.axis_index("y")
        my_z = lax.axis_index("z")
        peer = (my_x, 1 - my_y, my_z)

        barrier = pltpu.get_barrier_semaphore()
        pl.semaphore_signal(barrier, inc=1, device_id=peer,
                            device_id_type=pl.DeviceIdType.MESH)
        pl.semaphore_wait(barrier, 1)

        xbf_ref[...] = x_ref[...].astype(BF16)
        my_c = jnp.dot(
            xbf_ref[...], wdkv_ref[...].astype(BF16),
            preferred_element_type=F32,
        ).astype(BF16)

        def exchange(lo, hi):
            cc_ref[:, lo:hi] = my_c
            wukf_ref[lo:hi, :] = wuk_ref[...].astype(BF16)
            wuvf_ref[lo:hi, :] = wuv_ref[...].astype(BF16)
            rdmas = []
            srcs = [cc_ref.at[:, lo:hi], wukf_ref.at[lo:hi, :],
                    wuvf_ref.at[lo:hi, :]]
            for i, src in enumerate(srcs):
                rdma = pltpu.make_async_remote_copy(
                    src_ref=src, dst_ref=src,
                    send_sem=send_sems.at[i], recv_sem=recv_sems.at[i],
                    device_id=peer, device_id_type=pl.DeviceIdType.MESH,
                )
                rdma.start()
                rdmas.append(rdma)
            for rdma in rdmas:
                rdma.wait()

        @pl.when(my_y == 0)
        def _():
            exchange(0, DC_HALF)

        @pl.when(my_y == 1)
        def _():
            exchange(DC_HALF, 2 * DC_HALF)

    return pl.pallas_call(
        body,
        out_shape=[
            jax.ShapeDtypeStruct((BS, D), BF16),
            jax.ShapeDtypeStruct((BS, 2 * DC_HALF), BF16),
            jax.ShapeDtypeStruct((2 * DC_HALF, D), BF16),
            jax.ShapeDtypeStruct((2 * DC_HALF, D), BF16),
        ],
        in_specs=[pl.BlockSpec(memory_space=pltpu.VMEM)] * 4,
        out_specs=[pl.BlockSpec(memory_space=pltpu.VMEM)] * 4,
        scratch_shapes=[
            pltpu.SemaphoreType.DMA((3,)),
            pltpu.SemaphoreType.DMA((3,)),
        ],
        compiler_params=pltpu.CompilerParams(collective_id=0),
    )(x2d, Wdkv, Wuk, Wuv)


def _kv(cc, Wukf, Wuvf):
    nb = 8
    blk = D // nb
    dc = 2 * DC_HALF

    def body(cc_ref, wuk_ref, wuv_ref, k_ref, v_ref):
        c_ = cc_ref[...]
        k_ref[...] = jnp.dot(c_, wuk_ref[...],
                             preferred_element_type=F32).astype(BF16)
        v_ref[...] = jnp.dot(c_, wuv_ref[...],
                             preferred_element_type=F32).astype(BF16)

    return pl.pallas_call(
        body,
        grid=(nb,),
        in_specs=[
            pl.BlockSpec((BS, dc), lambda j: (0, 0)),
            pl.BlockSpec((dc, blk), lambda j: (0, j)),
            pl.BlockSpec((dc, blk), lambda j: (0, j)),
        ],
        out_specs=[
            pl.BlockSpec((BS, blk), lambda j: (0, j)),
            pl.BlockSpec((BS, blk), lambda j: (0, j)),
        ],
        out_shape=[
            jax.ShapeDtypeStruct((BS, D), BF16),
            jax.ShapeDtypeStruct((BS, D), BF16),
        ],
    )(cc, Wukf, Wuvf)


def _matmul(a_bf, w, n_blk, out_dtype):
    m, k = a_bf.shape
    _, n = w.shape
    nb = n // n_blk

    def body(a_ref, w_ref, o_ref):
        o_ref[...] = jnp.dot(
            a_ref[...], w_ref[...].astype(BF16),
            preferred_element_type=F32,
        ).astype(out_dtype)

    return pl.pallas_call(
        body,
        grid=(nb,),
        in_specs=[
            pl.BlockSpec((m, k), lambda j: (0, 0)),
            pl.BlockSpec((k, n_blk), lambda j: (0, j)),
        ],
        out_specs=pl.BlockSpec((m, n_blk), lambda j: (0, j)),
        out_shape=jax.ShapeDtypeStruct((m, n), out_dtype),
    )(a_bf, w)


def _kr_proj(xbf, Wkr):

    def body(x_ref, w_ref, o_ref):
        o_ref[...] = jnp.dot(x_ref[...], w_ref[...].astype(BF16),
                             preferred_element_type=F32).astype(BF16)

    return pl.pallas_call(
        body,
        in_specs=[pl.BlockSpec(memory_space=pltpu.VMEM)] * 2,
        out_specs=pl.BlockSpec(memory_space=pltpu.VMEM),
        out_shape=jax.ShapeDtypeStruct((BS, Dr), BF16),
    )(xbf, Wkr)


def _attention(Q2, K2, V2, Qr2, Kr):

    def body(q_ref, k_ref, v_ref, qr_ref, kr_ref, o_ref):
        dn = (((1,), (1,)), ((), ()))
        for b in range(B):
            r = slice(b * S, (b + 1) * S)
            kr = kr_ref[r, :]
            for hh in range(2):
                ch = slice(hh * Dh, (hh + 1) * Dh)
                cr = slice(hh * Dr, (hh + 1) * Dr)
                q = q_ref[r, ch]
                k = k_ref[r, ch]
                v = v_ref[r, ch]
                qr = qr_ref[r, cr]
                s = (lax.dot_general(q, k, dn, preferred_element_type=F32)
                     + lax.dot_general(qr, kr, dn, preferred_element_type=F32))
                s = s * SCALE
                m = jnp.max(s, axis=-1, keepdims=True)
                p = jnp.exp(s - m)
                p = p / jnp.sum(p, axis=-1, keepdims=True)
                o_ref[r, ch] = jnp.dot(p.astype(BF16), v,
                                       preferred_element_type=F32).astype(BF16)

    return pl.pallas_call(
        body,
        grid=(H // 2,),
        in_specs=[
            pl.BlockSpec((BS, 2 * Dh), lambda j: (0, j)),
            pl.BlockSpec((BS, 2 * Dh), lambda j: (0, j)),
            pl.BlockSpec((BS, 2 * Dh), lambda j: (0, j)),
            pl.BlockSpec((BS, 2 * Dr), lambda j: (0, j)),
            pl.BlockSpec((BS, Dr), lambda j: (0, 0)),
        ],
        out_specs=pl.BlockSpec((BS, 2 * Dh), lambda j: (0, j)),
        out_shape=jax.ShapeDtypeStruct((BS, D), BF16),
    )(Q2, K2, V2, Qr2, Kr)


def kernel(x, Wdkv, Wuk, Wuv, Wq, Wqr, Wkr, Wo):
    x2d = x.reshape(BS, D)

    xbf, cc, Wukf, Wuvf = _c_and_exchange(x2d, Wdkv, Wuk, Wuv)
    K, V = _kv(cc, Wukf, Wuvf)

    Q = _matmul(xbf, Wq, 512, BF16)
    Qr = _matmul(xbf, Wqr, 512, BF16)
    Kr = _kr_proj(xbf, Wkr)

    O = _attention(Q, K, V, Qr, Kr)

    out = _matmul(O, Wo, 512, F32)
    return out.reshape(B, S, D)


# device time: 203540 ns/iter; 3.4898x vs baseline; 1.1773x over previous
import jax
import jax.numpy as jnp
from jax import lax
from jax.experimental import pallas as pl
from jax.experimental.pallas import tpu as pltpu

B, S, H, Dh, Dr = 4, 256, 32, 128, 64
D = 4096
DC_HALF = 128
BS = B * S
SCALE = (Dh + Dr) ** -0.5

F32 = jnp.float32
BF16 = jnp.bfloat16


def _c_exchange_q(x2d, Wdkv, Wuk, Wuv, Wq):
    nb = 8
    blk = D // nb

    def body(x_ref, wdkv_ref, wuk_ref, wuv_ref, wq_ref,
             xbf_ref, cc_ref, wukf_ref, wuvf_ref, q_ref,
             send_sems, recv_sems):
        j = pl.program_id(0)
        my_x = lax.axis_index("x")
        my_y = lax.axis_index("y")
        my_z = lax.axis_index("z")
        peer = (my_x, 1 - my_y, my_z)

        def make_rdmas(lo, hi):
            srcs = [cc_ref.at[:, lo:hi], wukf_ref.at[lo:hi, :],
                    wuvf_ref.at[lo:hi, :]]
            return [
                pltpu.make_async_remote_copy(
                    src_ref=src, dst_ref=src,
                    send_sem=send_sems.at[i], recv_sem=recv_sems.at[i],
                    device_id=peer, device_id_type=pl.DeviceIdType.MESH,
                )
                for i, src in enumerate(srcs)
            ]

        def per_half(fn):
            @pl.when(my_y == 0)
            def _():
                fn(0, DC_HALF)

            @pl.when(my_y == 1)
            def _():
                fn(DC_HALF, 2 * DC_HALF)

        @pl.when(j == 0)
        def _():
            barrier = pltpu.get_barrier_semaphore()
            pl.semaphore_signal(barrier, inc=1, device_id=peer,
                                device_id_type=pl.DeviceIdType.MESH)
            pl.semaphore_wait(barrier, 1)

            xbf_ref[...] = x_ref[...].astype(BF16)
            my_c = jnp.dot(
                xbf_ref[...], wdkv_ref[...].astype(BF16),
                preferred_element_type=F32,
            ).astype(BF16)

            def fill_and_send(lo, hi):
                cc_ref[:, lo:hi] = my_c
                wukf_ref[lo:hi, :] = wuk_ref[...].astype(BF16)
                wuvf_ref[lo:hi, :] = wuv_ref[...].astype(BF16)
                for rdma in make_rdmas(lo, hi):
                    rdma.start()

            per_half(fill_and_send)

        q_ref[...] = (jnp.dot(xbf_ref[...], wq_ref[...].astype(BF16),
                              preferred_element_type=F32)
                      * SCALE).astype(BF16)

        @pl.when(j == nb - 1)
        def _():
            def wait_all(lo, hi):
                for rdma in make_rdmas(lo, hi):
                    rdma.wait()

            per_half(wait_all)

    return pl.pallas_call(
        body,
        grid=(nb,),
        in_specs=[
            pl.BlockSpec((BS, D), lambda j: (0, 0)),
            pl.BlockSpec((D, DC_HALF), lambda j: (0, 0)),
            pl.BlockSpec((DC_HALF, D), lambda j: (0, 0)),
            pl.BlockSpec((DC_HALF, D), lambda j: (0, 0)),
            pl.BlockSpec((D, blk), lambda j: (0, j)),
        ],
        out_specs=[
            pl.BlockSpec((BS, D), lambda j: (0, 0)),
            pl.BlockSpec((BS, 2 * DC_HALF), lambda j: (0, 0)),
            pl.BlockSpec((2 * DC_HALF, D), lambda j: (0, 0)),
            pl.BlockSpec((2 * DC_HALF, D), lambda j: (0, 0)),
            pl.BlockSpec((BS, blk), lambda j: (0, j)),
        ],
        out_shape=[
            jax.ShapeDtypeStruct((BS, D), BF16),
            jax.ShapeDtypeStruct((BS, 2 * DC_HALF), BF16),
            jax.ShapeDtypeStruct((2 * DC_HALF, D), BF16),
            jax.ShapeDtypeStruct((2 * DC_HALF, D), BF16),
            jax.ShapeDtypeStruct((BS, D), BF16),
        ],
        scratch_shapes=[
            pltpu.SemaphoreType.DMA((3,)),
            pltpu.SemaphoreType.DMA((3,)),
        ],
        compiler_params=pltpu.CompilerParams(
            collective_id=0, vmem_limit_bytes=100 * 1024 * 1024,
        ),
    )(x2d, Wdkv, Wuk, Wuv, Wq)


def _kv(cc, Wukf, Wuvf):
    nb = 8
    blk = D // nb
    dc = 2 * DC_HALF

    def body(cc_ref, wuk_ref, wuv_ref, k_ref, v_ref):
        c_ = cc_ref[...]
        k_ref[...] = jnp.dot(c_, wuk_ref[...],
                             preferred_element_type=F32).astype(BF16)
        v_ref[...] = jnp.dot(c_, wuv_ref[...],
                             preferred_element_type=F32).astype(BF16)

    return pl.pallas_call(
        body,
        grid=(nb,),
        in_specs=[
            pl.BlockSpec((BS, dc), lambda j: (0, 0)),
            pl.BlockSpec((dc, blk), lambda j: (0, j)),
            pl.BlockSpec((dc, blk), lambda j: (0, j)),
        ],
        out_specs=[
            pl.BlockSpec((BS, blk), lambda j: (0, j)),
            pl.BlockSpec((BS, blk), lambda j: (0, j)),
        ],
        out_shape=[
            jax.ShapeDtypeStruct((BS, D), BF16),
            jax.ShapeDtypeStruct((BS, D), BF16),
        ],
    )(cc, Wukf, Wuvf)


def _matmul(a_bf, w, n_blk, out_dtype, scale=None):
    m, k = a_bf.shape
    _, n = w.shape
    nb = n // n_blk

    def body(a_ref, w_ref, o_ref):
        r = jnp.dot(a_ref[...], w_ref[...].astype(BF16),
                    preferred_element_type=F32)
        if scale is not None:
            r = r * scale
        o_ref[...] = r.astype(out_dtype)

    return pl.pallas_call(
        body,
        grid=(nb,),
        in_specs=[
            pl.BlockSpec((m, k), lambda j: (0, 0)),
            pl.BlockSpec((k, n_blk), lambda j: (0, j)),
        ],
        out_specs=pl.BlockSpec((m, n_blk), lambda j: (0, j)),
        out_shape=jax.ShapeDtypeStruct((m, n), out_dtype),
    )(a_bf, w)


def _kr_proj(xbf, Wkr):

    def body(x_ref, w_ref, o_ref):
        o_ref[...] = jnp.dot(x_ref[...], w_ref[...].astype(BF16),
                             preferred_element_type=F32).astype(BF16)

    return pl.pallas_call(
        body,
        in_specs=[pl.BlockSpec(memory_space=pltpu.VMEM)] * 2,
        out_specs=pl.BlockSpec(memory_space=pltpu.VMEM),
        out_shape=jax.ShapeDtypeStruct((BS, Dr), BF16),
    )(xbf, Wkr)


def _attention(Q2, K2, V2, Qr2, Kr):

    def body(q_ref, k_ref, v_ref, qr_ref, kr_ref, o_ref):
        dn = (((1,), (1,)), ((), ()))
        for b in range(B):
            r = slice(b * S, (b + 1) * S)
            kr = kr_ref[r, :]
            for hh in range(2):
                ch = slice(hh * Dh, (hh + 1) * Dh)
                cr = slice(hh * Dr, (hh + 1) * Dr)
                q = q_ref[r, ch]
                k = k_ref[r, ch]
                v = v_ref[r, ch]
                qr = qr_ref[r, cr]
                s = (lax.dot_general(q, k, dn, preferred_element_type=F32)
                     + lax.dot_general(qr, kr, dn, preferred_element_type=F32))
                p = jnp.exp(s)
                p = p / jnp.sum(p, axis=-1, keepdims=True)
                o_ref[r, ch] = jnp.dot(p.astype(BF16), v,
                                       preferred_element_type=F32).astype(BF16)

    return pl.pallas_call(
        body,
        grid=(H // 2,),
        in_specs=[
            pl.BlockSpec((BS, 2 * Dh), lambda j: (0, j)),
            pl.BlockSpec((BS, 2 * Dh), lambda j: (0, j)),
            pl.BlockSpec((BS, 2 * Dh), lambda j: (0, j)),
            pl.BlockSpec((BS, 2 * Dr), lambda j: (0, j)),
            pl.BlockSpec((BS, Dr), lambda j: (0, 0)),
        ],
        out_specs=pl.BlockSpec((BS, 2 * Dh), lambda j: (0, j)),
        out_shape=jax.ShapeDtypeStruct((BS, D), BF16),
    )(Q2, K2, V2, Qr2, Kr)


def kernel(x, Wdkv, Wuk, Wuv, Wq, Wqr, Wkr, Wo):
    x2d = x.reshape(BS, D)

    xbf, cc, Wukf, Wuvf, Q = _c_exchange_q(x2d, Wdkv, Wuk, Wuv, Wq)
    K, V = _kv(cc, Wukf, Wuvf)

    Qr = _matmul(xbf, Wqr, 512, BF16, SCALE)
    Kr = _kr_proj(xbf, Wkr)

    O = _attention(Q, K, V, Qr, Kr)

    out = _matmul(O, Wo, 512, F32)
    return out.reshape(B, S, D)
